# baseline (speedup 1.0000x reference)
import functools

import jax
import jax.numpy as jnp
from jax import lax
from jax.experimental import pallas as pl
from jax.experimental.pallas import tpu as pltpu
from jax.experimental.pallas import tpu_sc as plsc

_D = 1000
_B = 1024 * 50
_NC = 2
_NS = 16
_NW = _NC * _NS
_BPW = _B // _NW
_CHUNK = 64
_NCHUNK = _BPW // _CHUNK

_mesh = plsc.VectorSubcoreMesh(core_axis_name="c", subcore_axis_name="s")


@functools.partial(
    pl.kernel,
    mesh=_mesh,
    out_type=jax.ShapeDtypeStruct((_B, _D), jnp.float32),
    scratch_types=[
        pltpu.VMEM((_NCHUNK, _CHUNK), jnp.int32),
        pltpu.VMEM((_CHUNK, _D), jnp.float32),
        pltpu.SemaphoreType.DMA,
    ],
    compiler_params=pltpu.CompilerParams(use_tc_tiling_on_sc=False),
)
def _gather_rows(table_hbm, idx_hbm, out_hbm, idx_v, rows_v, sem):
    wid = lax.axis_index("s") * _NC + lax.axis_index("c")
    pltpu.sync_copy(idx_hbm.at[wid], idx_v)
    base = wid * _BPW

    def step(c, carry):
        pltpu.async_copy(table_hbm.at[idx_v.at[c]], rows_v, sem).wait()
        pltpu.sync_copy(rows_v, out_hbm.at[pl.ds(base + c * _CHUNK, _CHUNK)])
        return carry

    lax.fori_loop(0, _NCHUNK, step, 0)


def kernel(x, table):
    idx = x.reshape(-1).astype(jnp.int32).reshape(_NW, _NCHUNK, _CHUNK)
    return _gather_rows(table, idx)


# speedup vs baseline: 1.0231x; 1.0231x over previous
import functools

import jax
import jax.numpy as jnp
from jax import lax
from jax.experimental import pallas as pl
from jax.experimental.pallas import tpu as pltpu
from jax.experimental.pallas import tpu_sc as plsc

_D = 1000
_B = 1024 * 50
_NC = 2
_NS = 16
_NW = _NC * _NS
_BPW = _B // _NW
_CHUNK = 32
_NCHUNK = _BPW // _CHUNK

_mesh = plsc.VectorSubcoreMesh(core_axis_name="c", subcore_axis_name="s")


@functools.partial(
    pl.kernel,
    mesh=_mesh,
    out_type=jax.ShapeDtypeStruct((_B, _D), jnp.float32),
    scratch_types=[
        pltpu.VMEM((_NCHUNK, _CHUNK), jnp.int32),
        pltpu.VMEM((2, _CHUNK, _D), jnp.float32),
        pltpu.SemaphoreType.DMA,
        pltpu.SemaphoreType.DMA,
        pltpu.SemaphoreType.DMA,
        pltpu.SemaphoreType.DMA,
    ],
    compiler_params=pltpu.CompilerParams(use_tc_tiling_on_sc=False),
)
def _gather_rows(table_hbm, idx_hbm, out_hbm, idx_v, rows_v, g0, g1, s0, s1):
    wid = lax.axis_index("s") * _NC + lax.axis_index("c")
    pltpu.sync_copy(idx_hbm.at[wid], idx_v)
    base = wid * _BPW
    gs = (g0, g1)
    ss = (s0, s1)

    def start_gather(c, b):
        pltpu.async_copy(table_hbm.at[idx_v.at[c]], rows_v.at[b], gs[b])

    def wait_gather(b):
        pltpu.make_async_copy(
            table_hbm.at[pl.ds(0, _CHUNK)], rows_v.at[b], gs[b]).wait()

    def start_scatter(c, b):
        pltpu.async_copy(
            rows_v.at[b], out_hbm.at[pl.ds(base + c * _CHUNK, _CHUNK)], ss[b])

    def wait_scatter(b):
        pltpu.make_async_copy(
            rows_v.at[b], out_hbm.at[pl.ds(base, _CHUNK)], ss[b]).wait()

    start_gather(0, 0)
    start_gather(1, 1)

    def pair(p, carry):
        c0 = 2 * p
        for b in range(2):
            c = c0 + b
            wait_gather(b)
            start_scatter(c, b)

            @pl.when(c + 2 < _NCHUNK)
            def _():
                wait_scatter(b)
                start_gather(c + 2, b)

        return carry

    lax.fori_loop(0, _NCHUNK // 2, pair, 0)
    wait_scatter(0)
    wait_scatter(1)


def kernel(x, table):
    idx = x.reshape(-1).astype(jnp.int32).reshape(_NW, _NCHUNK, _CHUNK)
    return _gather_rows(table, idx)


# speedup vs baseline: 1.5199x; 1.4855x over previous
import functools

import jax
import jax.numpy as jnp
from jax import lax
from jax.experimental import pallas as pl
from jax.experimental.pallas import tpu as pltpu
from jax.experimental.pallas import tpu_sc as plsc

_D = 1000
_DA = 896
_DB = _D - _DA
_B = 1024 * 50
_NC = 2
_NS = 16
_NW = _NC * _NS
_BPW = _B // _NW
_CHUNK = 32
_NCHUNK = _BPW // _CHUNK

_mesh = plsc.VectorSubcoreMesh(core_axis_name="c", subcore_axis_name="s")


@functools.partial(
    pl.kernel,
    mesh=_mesh,
    out_type=jax.ShapeDtypeStruct((_B, _D), jnp.float32),
    scratch_types=[
        pltpu.VMEM((_NCHUNK, _CHUNK), jnp.int32),
        pltpu.VMEM((2, _CHUNK, _D), jnp.float32),
        pltpu.VMEM((2, _CHUNK, 128), jnp.float32),
        pltpu.SemaphoreType.DMA,
        pltpu.SemaphoreType.DMA,
        pltpu.SemaphoreType.DMA,
        pltpu.SemaphoreType.DMA,
        pltpu.SemaphoreType.DMA,
        pltpu.SemaphoreType.DMA,
    ],
)
def _gather_rows(ta_hbm, tb_hbm, idx_hbm, out_hbm, idx_v, rows_v, tail_v,
                 ga0, ga1, gb0, gb1, s0, s1):
    wid = lax.axis_index("s") * _NC + lax.axis_index("c")
    pltpu.sync_copy(idx_hbm.at[wid], idx_v)
    base = wid * _BPW
    gas = (ga0, ga1)
    gbs = (gb0, gb1)
    ss = (s0, s1)

    def start_gathers(c, b):
        pltpu.async_copy(ta_hbm.at[idx_v.at[c]],
                         rows_v.at[b].at[:, pl.ds(0, _DA)], gas[b])
        pltpu.async_copy(tb_hbm.at[idx_v.at[c]], tail_v.at[b], gbs[b])

    def wait_gathers(b):
        pltpu.make_async_copy(ta_hbm.at[pl.ds(0, _CHUNK)],
                              rows_v.at[b].at[:, pl.ds(0, _DA)], gas[b]).wait()
        pltpu.make_async_copy(tb_hbm.at[pl.ds(0, _CHUNK)],
                              tail_v.at[b], gbs[b]).wait()

    def stitch_tail(b):
        for r in range(_CHUNK):
            for k in range(6):
                rows_v[b, r, pl.ds(_DA + 16 * k, 16)] = (
                    tail_v[b, r, pl.ds(16 * k, 16)])
            rows_v[b, r, pl.ds(_D - 16, 16)] = (
                tail_v[b, r, pl.ds(_DB - 16, 16)])

    def start_scatter(c, b):
        pltpu.async_copy(
            rows_v.at[b], out_hbm.at[pl.ds(base + c * _CHUNK, _CHUNK)], ss[b])

    def wait_scatter(b):
        pltpu.make_async_copy(
            rows_v.at[b], out_hbm.at[pl.ds(base, _CHUNK)], ss[b]).wait()

    start_gathers(0, 0)
    start_gathers(1, 1)

    def pair(p, carry):
        c0 = 2 * p
        for b in range(2):
            c = c0 + b
            wait_gathers(b)
            stitch_tail(b)
            start_scatter(c, b)

            @pl.when(c + 2 < _NCHUNK)
            def _():
                wait_scatter(b)
                start_gathers(c + 2, b)

        return carry

    lax.fori_loop(0, _NCHUNK // 2, pair, 0)
    wait_scatter(0)
    wait_scatter(1)


def kernel(x, table):
    idx = x.reshape(-1).astype(jnp.int32).reshape(_NW, _NCHUNK, _CHUNK)
    table_a = table[:, :_DA]
    table_b = jnp.pad(table[:, _DA:], ((0, 0), (0, 128 - _DB)))
    return _gather_rows(table_a, table_b, idx)


# speedup vs baseline: 1.7108x; 1.1256x over previous
import functools

import jax
import jax.numpy as jnp
from jax import lax
from jax.experimental import pallas as pl
from jax.experimental.pallas import tpu as pltpu
from jax.experimental.pallas import tpu_sc as plsc

_D = 1000
_DP = 1024
_B = 1024 * 50
_NC = 2
_NS = 16
_NW = _NC * _NS
_BPW = _B // _NW
_CHUNK = 32
_NCHUNK = _BPW // _CHUNK

_mesh = plsc.VectorSubcoreMesh(core_axis_name="c", subcore_axis_name="s")


@functools.partial(
    pl.kernel,
    mesh=_mesh,
    out_type=jax.ShapeDtypeStruct((_B, _DP), jnp.float32),
    scratch_types=[
        pltpu.VMEM((_NCHUNK, _CHUNK), jnp.int32),
        pltpu.VMEM((_CHUNK, _DP), jnp.float32),
        pltpu.VMEM((_CHUNK, _DP), jnp.float32),
        pltpu.SemaphoreType.DMA,
        pltpu.SemaphoreType.DMA,
        pltpu.SemaphoreType.DMA,
        pltpu.SemaphoreType.DMA,
    ],
)
def _gather_rows(t_hbm, idx_hbm, out_hbm, idx_v, rows_v0, rows_v1,
                 g0, g1, s0, s1):
    wid = lax.axis_index("s") * _NC + lax.axis_index("c")
    pltpu.sync_copy(idx_hbm.at[wid], idx_v)
    base = wid * _BPW
    rows = (rows_v0, rows_v1)
    gs = (g0, g1)
    ss = (s0, s1)

    def start_gather(c, b):
        pltpu.async_copy(t_hbm.at[idx_v.at[c]], rows[b], gs[b])

    def wait_gather(b):
        pltpu.make_async_copy(t_hbm.at[pl.ds(0, _CHUNK)], rows[b],
                              gs[b]).wait()

    def start_scatter(c, b):
        pltpu.async_copy(rows[b], out_hbm.at[pl.ds(base + c * _CHUNK, _CHUNK)],
                         ss[b])

    def wait_scatter(b):
        pltpu.make_async_copy(rows[b], out_hbm.at[pl.ds(base, _CHUNK)],
                              ss[b]).wait()

    start_gather(0, 0)
    start_gather(1, 1)

    def pair(p, carry):
        c0 = 2 * p
        for b in range(2):
            c = c0 + b
            wait_gather(b)
            start_scatter(c, b)

            @pl.when(c + 2 < _NCHUNK)
            def _():
                wait_scatter(b)
                start_gather(c + 2, b)

        return carry

    lax.fori_loop(0, _NCHUNK // 2, pair, 0)
    wait_scatter(0)
    wait_scatter(1)


def kernel(x, table):
    idx = x.reshape(-1).astype(jnp.int32).reshape(_NW, _NCHUNK, _CHUNK)
    table_pad = jnp.pad(table, ((0, 0), (0, _DP - _D)))
    return _gather_rows(table_pad, idx)[:, :_D]
